# Initial kernel scaffold; baseline (speedup 1.0000x reference)
import functools

import jax
import jax.numpy as jnp
from jax import lax
from jax.experimental import pallas as pl
from jax.experimental.pallas import tpu as pltpu
from jax.experimental.pallas import tpu_sc as plsc

NC = 2
NS = 16
LANE = 16
CHUNK = 128


def _cdiv(a, b):
    return (a + b - 1) // b


def _dinv_from_degw(degw):
    s = (jnp.sum(degw[0], axis=-1, keepdims=True)
         + jnp.sum(degw[1], axis=-1, keepdims=True))
    deg = s * (1.0 / LANE)
    return jnp.where(deg > 0, lax.rsqrt(jnp.where(deg > 0, deg, 1.0)), 0.0)


def _tc_first_body(x_ref, w_ref, degw_ref, o_ref):
    dinv = _dinv_from_degw(degw_ref)
    t = lax.dot_general(x_ref[...], w_ref[...], (((1,), (1,)), ((), ())),
                        preferred_element_type=jnp.float32)
    o_ref[...] = t * dinv


def _tc_mid_body(agg_ref, degw_ref, b_ref, w_ref, o_ref):
    dinv = _dinv_from_degw(degw_ref)
    h = (agg_ref[0] + agg_ref[1]) * dinv + b_ref[...]
    h = jnp.maximum(h, 0.0)
    t = lax.dot_general(h, w_ref[...], (((1,), (1,)), ((), ())),
                        preferred_element_type=jnp.float32)
    o_ref[...] = t * dinv


def _tc_last_body(agg_ref, degw_ref, b_ref, o_ref):
    dinv = _dinv_from_degw(degw_ref)
    o_ref[...] = (agg_ref[0] + agg_ref[1]) * dinv + b_ref[...]


def _row_blocks(n):
    for blk in (400, 200, 100, 50, 25, 10, 8):
        if n % blk == 0:
            return blk
    return n


def _make_sc_deg(n_pad, rw):
    mesh = plsc.VectorSubcoreMesh(core_axis_name="c", subcore_axis_name="s")
    zrows = n_pad // NS
    orows = (n_pad - LANE) // NS

    def body(dst_hbm, zeros_hbm, ones_hbm, out_hbm, dst_all, ones_v, accd):
        cid = lax.axis_index("c")
        sid = lax.axis_index("s")
        wid = cid * NS + sid
        pltpu.sync_copy(zeros_hbm.at[pl.ds(sid * zrows, zrows)],
                        accd.at[pl.ds(sid * zrows, zrows)])
        pltpu.sync_copy(ones_hbm, ones_v)
        pltpu.sync_copy(dst_hbm.at[pl.ds(wid * rw, rw)], dst_all)
        plsc.subcore_barrier()

        def step(j, c):
            pltpu.sync_copy(ones_v, accd.at[dst_all.at[j]], add=True)
            return c

        lax.fori_loop(0, rw, step, 0)
        plsc.subcore_barrier()
        pltpu.sync_copy(accd.at[pl.ds(sid * orows, orows)],
                        out_hbm.at[cid, pl.ds(sid * orows, orows)])

    return pl.kernel(
        body,
        out_type=jax.ShapeDtypeStruct((NC, n_pad - LANE, LANE), jnp.float32),
        mesh=mesh,
        scratch_types=[
            pltpu.VMEM((rw, CHUNK), jnp.int32),
            pltpu.VMEM((CHUNK, LANE), jnp.float32),
            pltpu.VMEM_SHARED((n_pad, LANE), jnp.float32),
        ],
    )


def _make_sc_agg(n, n_pad, d, rw):
    mesh = plsc.VectorSubcoreMesh(core_axis_name="c", subcore_axis_name="s")
    zrows = n_pad // NS
    orows = n // NS

    def body(t_hbm, src_hbm, dst_hbm, zeros_hbm, out_hbm,
             src_all, dst_all, buf, acc, sem):
        cid = lax.axis_index("c")
        sid = lax.axis_index("s")
        wid = cid * NS + sid
        pltpu.sync_copy(zeros_hbm.at[pl.ds(sid * zrows, zrows)],
                        acc.at[pl.ds(sid * zrows, zrows)])
        pltpu.sync_copy(src_hbm.at[pl.ds(wid * rw, rw)], src_all)
        pltpu.sync_copy(dst_hbm.at[pl.ds(wid * rw, rw)], dst_all)
        plsc.subcore_barrier()

        def step(j, c):
            pltpu.async_copy(t_hbm.at[src_all.at[j]], buf, sem).wait()
            pltpu.sync_copy(buf, acc.at[dst_all.at[j]], add=True)
            return c

        lax.fori_loop(0, rw, step, 0)
        plsc.subcore_barrier()
        pltpu.sync_copy(acc.at[pl.ds(sid * orows, orows)],
                        out_hbm.at[cid, pl.ds(sid * orows, orows)])

    return pl.kernel(
        body,
        out_type=jax.ShapeDtypeStruct((NC, n, d), jnp.float32),
        mesh=mesh,
        scratch_types=[
            pltpu.VMEM((rw, CHUNK), jnp.int32),
            pltpu.VMEM((rw, CHUNK), jnp.int32),
            pltpu.VMEM((CHUNK, d), jnp.float32),
            pltpu.VMEM_SHARED((n_pad, d), jnp.float32),
            pltpu.SemaphoreType.DMA,
        ],
    )


def kernel(x, edge_index, W1, b1, W2, b2):
    n, d = x.shape
    e = edge_index.shape[1]
    d_hid = W1.shape[0]
    d_out = W2.shape[0]

    nw = NC * NS
    rw = _cdiv(e, CHUNK * nw)
    if rw % 2:
        rw += 1
    e_pad = nw * rw * CHUNK
    n_pad = (_cdiv(n, NS) * NS) + LANE

    src = edge_index[0].astype(jnp.int32)
    dst = edge_index[1].astype(jnp.int32)
    src2d = jnp.pad(src, (0, e_pad - e)).reshape(nw * rw, CHUNK)
    dst2d = jnp.pad(dst, (0, e_pad - e),
                    constant_values=n_pad - LANE).reshape(nw * rw, CHUNK)

    zeros_w = jnp.zeros((n_pad, LANE), jnp.float32)
    zeros_d = jnp.zeros((n_pad, d_hid), jnp.float32)
    ones_w = jnp.ones((CHUNK, LANE), jnp.float32)

    degw = _make_sc_deg(n_pad, rw)(dst2d, zeros_w, ones_w)
    degw = degw[:, :n, :]

    sc_agg = _make_sc_agg(n, n_pad, d_hid, rw)

    blk = _row_blocks(n)
    grid = (n // blk,)
    degw_spec = pl.BlockSpec((NC, blk, LANE), lambda i: (0, i, 0))
    row_spec = pl.BlockSpec((blk, d), lambda i: (i, 0))
    w_spec = pl.BlockSpec((d_hid, d), lambda i: (0, 0))
    b_spec = pl.BlockSpec((1, d_hid), lambda i: (0, 0))
    agg_spec = pl.BlockSpec((NC, blk, d_hid), lambda i: (0, i, 0))

    t1 = pl.pallas_call(
        _tc_first_body,
        grid=grid,
        in_specs=[row_spec, w_spec, degw_spec],
        out_specs=pl.BlockSpec((blk, d_hid), lambda i: (i, 0)),
        out_shape=jax.ShapeDtypeStruct((n, d_hid), jnp.float32),
    )(x, W1, degw)

    agg1 = sc_agg(t1, src2d, dst2d, zeros_d)

    t2 = pl.pallas_call(
        _tc_mid_body,
        grid=grid,
        in_specs=[agg_spec, degw_spec, b_spec,
                  pl.BlockSpec((d_out, d_hid), lambda i: (0, 0))],
        out_specs=pl.BlockSpec((blk, d_out), lambda i: (i, 0)),
        out_shape=jax.ShapeDtypeStruct((n, d_out), jnp.float32),
    )(agg1, degw, b1.reshape(1, d_hid), W2)

    agg2 = sc_agg(t2, src2d, dst2d, zeros_d)

    out = pl.pallas_call(
        _tc_last_body,
        grid=grid,
        in_specs=[agg_spec, degw_spec, b_spec],
        out_specs=pl.BlockSpec((blk, d_out), lambda i: (i, 0)),
        out_shape=jax.ShapeDtypeStruct((n, d_out), jnp.float32),
    )(agg2, degw, b2.reshape(1, d_out))

    return out


# speedup vs baseline: 6.4115x; 6.4115x over previous
import functools

import jax
import jax.numpy as jnp
from jax import lax
from jax.experimental import pallas as pl
from jax.experimental.pallas import tpu as pltpu
from jax.experimental.pallas import tpu_sc as plsc

NC = 2
NS = 16
LANE = 16
CHUNK = 128


def _cdiv(a, b):
    return (a + b - 1) // b


def _dinv_from_degw(degw):
    w = degw.shape[-1]
    s = (jnp.sum(degw[0], axis=-1, keepdims=True)
         + jnp.sum(degw[1], axis=-1, keepdims=True))
    deg = s * (1.0 / w)
    return jnp.where(deg > 0, lax.rsqrt(jnp.where(deg > 0, deg, 1.0)), 0.0)


def _tc_first_body(x_ref, w_ref, degw_ref, o_ref):
    dinv = _dinv_from_degw(degw_ref)
    t = lax.dot_general(x_ref[...], w_ref[...], (((1,), (1,)), ((), ())),
                        preferred_element_type=jnp.float32)
    o_ref[...] = t * dinv


def _tc_mid_body(agg_ref, degw_ref, b_ref, w_ref, o_ref):
    dinv = _dinv_from_degw(degw_ref)
    h = (agg_ref[0] + agg_ref[1]) * dinv + b_ref[...]
    h = jnp.maximum(h, 0.0)
    t = lax.dot_general(h, w_ref[...], (((1,), (1,)), ((), ())),
                        preferred_element_type=jnp.float32)
    o_ref[...] = t * dinv


def _tc_last_body(agg_ref, degw_ref, b_ref, o_ref):
    dinv = _dinv_from_degw(degw_ref)
    o_ref[...] = (agg_ref[0] + agg_ref[1]) * dinv + b_ref[...]


def _row_blocks(n):
    for blk in (400, 200, 100, 50, 25, 10, 8):
        if n % blk == 0:
            return blk
    return n


def _make_sc_deg(n_pad, d, rw):
    mesh = plsc.VectorSubcoreMesh(core_axis_name="c", subcore_axis_name="s")
    zrows = n_pad // NS
    orows = n_pad // NS

    def body(dst_hbm, zeros_hbm, ones_hbm, out_hbm, dst_all, ones_v, accd):
        cid = lax.axis_index("c")
        sid = lax.axis_index("s")
        wid = cid * NS + sid
        pltpu.sync_copy(zeros_hbm.at[pl.ds(sid * zrows, zrows)],
                        accd.at[pl.ds(sid * zrows, zrows)])
        pltpu.sync_copy(ones_hbm, ones_v)
        pltpu.sync_copy(dst_hbm.at[pl.ds(wid * rw, rw)], dst_all)
        plsc.subcore_barrier()

        def step(j, c):
            pltpu.sync_copy(ones_v, accd.at[dst_all.at[j]], add=True)
            return c

        lax.fori_loop(0, rw, step, 0)
        plsc.subcore_barrier()
        pltpu.sync_copy(accd.at[pl.ds(sid * orows, orows)],
                        out_hbm.at[cid, pl.ds(sid * orows, orows)])

    return pl.kernel(
        body,
        out_type=jax.ShapeDtypeStruct((NC, n_pad, d), jnp.float32),
        mesh=mesh,
        scratch_types=[
            pltpu.VMEM((rw, CHUNK), jnp.int32),
            pltpu.VMEM((CHUNK, d), jnp.float32),
            pltpu.VMEM_SHARED((n_pad, d), jnp.float32),
        ],
    )


def _make_sc_agg(n_pad, d, rw):
    mesh = plsc.VectorSubcoreMesh(core_axis_name="c", subcore_axis_name="s")
    zrows = n_pad // NS
    orows = n_pad // NS

    def body(t_hbm, src_hbm, dst_hbm, zeros_hbm, out_hbm,
             src_all, dst_all, buf, acc, sem):
        cid = lax.axis_index("c")
        sid = lax.axis_index("s")
        wid = cid * NS + sid
        pltpu.sync_copy(zeros_hbm.at[pl.ds(sid * zrows, zrows)],
                        acc.at[pl.ds(sid * zrows, zrows)])
        pltpu.sync_copy(src_hbm.at[pl.ds(wid * rw, rw)], src_all)
        pltpu.sync_copy(dst_hbm.at[pl.ds(wid * rw, rw)], dst_all)
        plsc.subcore_barrier()

        def step(j, c):
            pltpu.async_copy(t_hbm.at[src_all.at[j]], buf, sem).wait()
            pltpu.sync_copy(buf, acc.at[dst_all.at[j]], add=True)
            return c

        lax.fori_loop(0, rw, step, 0)
        plsc.subcore_barrier()
        pltpu.sync_copy(acc.at[pl.ds(sid * orows, orows)],
                        out_hbm.at[cid, pl.ds(sid * orows, orows)])

    return pl.kernel(
        body,
        out_type=jax.ShapeDtypeStruct((NC, n_pad, d), jnp.float32),
        mesh=mesh,
        scratch_types=[
            pltpu.VMEM((rw, CHUNK), jnp.int32),
            pltpu.VMEM((rw, CHUNK), jnp.int32),
            pltpu.VMEM((CHUNK, d), jnp.float32),
            pltpu.VMEM_SHARED((n_pad, d), jnp.float32),
            pltpu.SemaphoreType.DMA,
        ],
    )


def kernel(x, edge_index, W1, b1, W2, b2):
    n, d = x.shape
    e = edge_index.shape[1]
    d_hid = W1.shape[0]
    d_out = W2.shape[0]

    nw = NC * NS
    rw = _cdiv(e, CHUNK * nw)
    if rw % 2:
        rw += 1
    e_pad = nw * rw * CHUNK
    n_pad = _cdiv(n + 1, 128) * 128

    src = edge_index[0].astype(jnp.int32)
    dst = edge_index[1].astype(jnp.int32)
    src2d = jnp.pad(src, (0, e_pad - e)).reshape(nw * rw, CHUNK)
    dst2d = jnp.pad(dst, (0, e_pad - e),
                    constant_values=n).reshape(nw * rw, CHUNK)

    zeros_d = jnp.zeros((n_pad, d_hid), jnp.float32)
    ones_d = jnp.ones((CHUNK, d_hid), jnp.float32)

    degw = _make_sc_deg(n_pad, d_hid, rw)(dst2d, zeros_d, ones_d)

    sc_agg = _make_sc_agg(n_pad, d_hid, rw)

    blk = _row_blocks(n)
    grid = (n // blk,)
    degw_spec = pl.BlockSpec((NC, blk, d_hid), lambda i: (0, i, 0))
    row_spec = pl.BlockSpec((blk, d), lambda i: (i, 0))
    w_spec = pl.BlockSpec((d_hid, d), lambda i: (0, 0))
    b_spec = pl.BlockSpec((1, d_hid), lambda i: (0, 0))
    agg_spec = pl.BlockSpec((NC, blk, d_hid), lambda i: (0, i, 0))

    t1 = pl.pallas_call(
        _tc_first_body,
        grid=grid,
        in_specs=[row_spec, w_spec, degw_spec],
        out_specs=pl.BlockSpec((blk, d_hid), lambda i: (i, 0)),
        out_shape=jax.ShapeDtypeStruct((n, d_hid), jnp.float32),
    )(x, W1, degw)

    agg1 = sc_agg(t1, src2d, dst2d, zeros_d)

    t2 = pl.pallas_call(
        _tc_mid_body,
        grid=grid,
        in_specs=[agg_spec, degw_spec, b_spec,
                  pl.BlockSpec((d_out, d_hid), lambda i: (0, 0))],
        out_specs=pl.BlockSpec((blk, d_out), lambda i: (i, 0)),
        out_shape=jax.ShapeDtypeStruct((n, d_out), jnp.float32),
    )(agg1, degw, b1.reshape(1, d_hid), W2)

    agg2 = sc_agg(t2, src2d, dst2d, zeros_d)

    out = pl.pallas_call(
        _tc_last_body,
        grid=grid,
        in_specs=[agg_spec, degw_spec, b_spec],
        out_specs=pl.BlockSpec((blk, d_out), lambda i: (i, 0)),
        out_shape=jax.ShapeDtypeStruct((n, d_out), jnp.float32),
    )(agg2, degw, b2.reshape(1, d_out))

    return out


# speedup vs baseline: 7.3306x; 1.1434x over previous
import functools

import jax
import jax.numpy as jnp
from jax import lax
from jax.experimental import pallas as pl
from jax.experimental.pallas import tpu as pltpu
from jax.experimental.pallas import tpu_sc as plsc

NC = 2
NS = 16
LANE = 16
CHUNK = 128


def _cdiv(a, b):
    return (a + b - 1) // b


def _dinv_from_degw(degw):
    w = degw.shape[-1]
    s = (jnp.sum(degw[0], axis=-1, keepdims=True)
         + jnp.sum(degw[1], axis=-1, keepdims=True))
    deg = s * (1.0 / w)
    return jnp.where(deg > 0, lax.rsqrt(jnp.where(deg > 0, deg, 1.0)), 0.0)


def _tc_first_body(x_ref, w_ref, degw_ref, o_ref):
    dinv = _dinv_from_degw(degw_ref)
    t = lax.dot_general(x_ref[...], w_ref[...], (((1,), (1,)), ((), ())),
                        preferred_element_type=jnp.float32)
    o_ref[...] = t * dinv


def _tc_mid_body(agg_ref, degw_ref, b_ref, w_ref, o_ref):
    dinv = _dinv_from_degw(degw_ref)
    h = (agg_ref[0] + agg_ref[1]) * dinv + b_ref[...]
    h = jnp.maximum(h, 0.0)
    t = lax.dot_general(h, w_ref[...], (((1,), (1,)), ((), ())),
                        preferred_element_type=jnp.float32)
    o_ref[...] = t * dinv


def _tc_last_body(agg_ref, degw_ref, b_ref, o_ref):
    dinv = _dinv_from_degw(degw_ref)
    o_ref[...] = (agg_ref[0] + agg_ref[1]) * dinv + b_ref[...]


def _row_blocks(n):
    for blk in (400, 200, 100, 50, 25, 10, 8):
        if n % blk == 0:
            return blk
    return n


def _make_sc_deg(n_pad, d, rw):
    mesh = plsc.VectorSubcoreMesh(core_axis_name="c", subcore_axis_name="s")
    zrows = n_pad // NS
    orows = n_pad // NS

    def body(dst_hbm, zeros_hbm, ones_hbm, out_hbm, dst_all, ones_v, accd):
        cid = lax.axis_index("c")
        sid = lax.axis_index("s")
        wid = cid * NS + sid
        pltpu.sync_copy(zeros_hbm.at[pl.ds(sid * zrows, zrows)],
                        accd.at[pl.ds(sid * zrows, zrows)])
        pltpu.sync_copy(ones_hbm, ones_v)
        pltpu.sync_copy(dst_hbm.at[pl.ds(wid * rw, rw)], dst_all)
        plsc.subcore_barrier()

        def step(j, c):
            pltpu.sync_copy(ones_v, accd.at[dst_all.at[j]], add=True)
            return c

        lax.fori_loop(0, rw, step, 0)
        plsc.subcore_barrier()
        pltpu.sync_copy(accd.at[pl.ds(sid * orows, orows)],
                        out_hbm.at[cid, pl.ds(sid * orows, orows)])

    return pl.kernel(
        body,
        out_type=jax.ShapeDtypeStruct((NC, n_pad, d), jnp.float32),
        mesh=mesh,
        scratch_types=[
            pltpu.VMEM((rw, CHUNK), jnp.int32),
            pltpu.VMEM((CHUNK, d), jnp.float32),
            pltpu.VMEM_SHARED((n_pad, d), jnp.float32),
        ],
    )


NBUF = 2
IB = 16


def _make_sc_agg(n_pad, d, rw):
    mesh = plsc.VectorSubcoreMesh(core_axis_name="c", subcore_axis_name="s")
    zrows = n_pad // NS
    orows = n_pad // NS

    def body(t_hbm, src_hbm, dst_hbm, zeros_hbm, out_hbm,
             src_ib, dst_ib, bufs, acc, sems):
        cid = lax.axis_index("c")
        sid = lax.axis_index("s")
        wid = cid * NS + sid
        pltpu.sync_copy(zeros_hbm.at[pl.ds(sid * zrows, zrows)],
                        acc.at[pl.ds(sid * zrows, zrows)])
        plsc.subcore_barrier()

        def block(g, c):
            base = wid * rw + g * IB
            pltpu.sync_copy(src_hbm.at[pl.ds(base, IB)], src_ib)
            pltpu.sync_copy(dst_hbm.at[pl.ds(base, IB)], dst_ib)
            for b in range(NBUF):
                pltpu.async_copy(t_hbm.at[src_ib.at[b]], bufs.at[b],
                                 sems.at[b])
            for j in range(IB):
                b = j % NBUF
                pltpu.make_async_copy(t_hbm.at[src_ib.at[j]], bufs.at[b],
                                      sems.at[b]).wait()
                pltpu.sync_copy(bufs.at[b], acc.at[dst_ib.at[j]], add=True)
                if j + NBUF < IB:
                    pltpu.async_copy(t_hbm.at[src_ib.at[j + NBUF]],
                                     bufs.at[b], sems.at[b])
            return c

        lax.fori_loop(0, rw // IB, block, 0)
        plsc.subcore_barrier()
        pltpu.sync_copy(acc.at[pl.ds(sid * orows, orows)],
                        out_hbm.at[cid, pl.ds(sid * orows, orows)])

    return pl.kernel(
        body,
        out_type=jax.ShapeDtypeStruct((NC, n_pad, d), jnp.float32),
        mesh=mesh,
        scratch_types=[
            pltpu.VMEM((IB, CHUNK), jnp.int32),
            pltpu.VMEM((IB, CHUNK), jnp.int32),
            pltpu.VMEM((NBUF, CHUNK, d), jnp.float32),
            pltpu.VMEM_SHARED((n_pad, d), jnp.float32),
            pltpu.SemaphoreType.DMA((NBUF,)),
        ],
    )


def kernel(x, edge_index, W1, b1, W2, b2):
    n, d = x.shape
    e = edge_index.shape[1]
    d_hid = W1.shape[0]
    d_out = W2.shape[0]

    nw = NC * NS
    rw = _cdiv(_cdiv(e, CHUNK * nw), IB) * IB
    e_pad = nw * rw * CHUNK
    n_pad = _cdiv(n + 1, 128) * 128

    src = edge_index[0].astype(jnp.int32)
    dst = edge_index[1].astype(jnp.int32)
    src2d = jnp.pad(src, (0, e_pad - e)).reshape(nw * rw, CHUNK)
    dst2d = jnp.pad(dst, (0, e_pad - e),
                    constant_values=n).reshape(nw * rw, CHUNK)

    zeros_d = jnp.zeros((n_pad, d_hid), jnp.float32)
    ones_d = jnp.ones((CHUNK, d_hid), jnp.float32)

    degw = _make_sc_deg(n_pad, d_hid, rw)(dst2d, zeros_d, ones_d)

    sc_agg = _make_sc_agg(n_pad, d_hid, rw)

    blk = _row_blocks(n)
    grid = (n // blk,)
    degw_spec = pl.BlockSpec((NC, blk, d_hid), lambda i: (0, i, 0))
    row_spec = pl.BlockSpec((blk, d), lambda i: (i, 0))
    w_spec = pl.BlockSpec((d_hid, d), lambda i: (0, 0))
    b_spec = pl.BlockSpec((1, d_hid), lambda i: (0, 0))
    agg_spec = pl.BlockSpec((NC, blk, d_hid), lambda i: (0, i, 0))

    t1 = pl.pallas_call(
        _tc_first_body,
        grid=grid,
        in_specs=[row_spec, w_spec, degw_spec],
        out_specs=pl.BlockSpec((blk, d_hid), lambda i: (i, 0)),
        out_shape=jax.ShapeDtypeStruct((n, d_hid), jnp.float32),
    )(x, W1, degw)

    agg1 = sc_agg(t1, src2d, dst2d, zeros_d)

    t2 = pl.pallas_call(
        _tc_mid_body,
        grid=grid,
        in_specs=[agg_spec, degw_spec, b_spec,
                  pl.BlockSpec((d_out, d_hid), lambda i: (0, 0))],
        out_specs=pl.BlockSpec((blk, d_out), lambda i: (i, 0)),
        out_shape=jax.ShapeDtypeStruct((n, d_out), jnp.float32),
    )(agg1, degw, b1.reshape(1, d_hid), W2)

    agg2 = sc_agg(t2, src2d, dst2d, zeros_d)

    out = pl.pallas_call(
        _tc_last_body,
        grid=grid,
        in_specs=[agg_spec, degw_spec, b_spec],
        out_specs=pl.BlockSpec((blk, d_out), lambda i: (i, 0)),
        out_shape=jax.ShapeDtypeStruct((n, d_out), jnp.float32),
    )(agg2, degw, b2.reshape(1, d_out))

    return out


# speedup vs baseline: 7.8660x; 1.0730x over previous
import functools

import jax
import jax.numpy as jnp
from jax import lax
from jax.experimental import pallas as pl
from jax.experimental.pallas import tpu as pltpu
from jax.experimental.pallas import tpu_sc as plsc

NC = 2
NS = 16
LANE = 16
CHUNK = 128


def _cdiv(a, b):
    return (a + b - 1) // b


def _dinv_from_degw(degw):
    w = degw.shape[-1]
    s = (jnp.sum(degw[0], axis=-1, keepdims=True)
         + jnp.sum(degw[1], axis=-1, keepdims=True))
    deg = s * (1.0 / w)
    return jnp.where(deg > 0, lax.rsqrt(jnp.where(deg > 0, deg, 1.0)), 0.0)


def _tc_first_body(x_ref, w_ref, degw_ref, o_ref):
    dinv = _dinv_from_degw(degw_ref)
    t = lax.dot_general(x_ref[...], w_ref[...], (((1,), (1,)), ((), ())),
                        preferred_element_type=jnp.float32)
    o_ref[...] = t * dinv


def _tc_mid_body(agg_ref, degw_ref, b_ref, w_ref, o_ref):
    dinv = _dinv_from_degw(degw_ref)
    h = (agg_ref[0] + agg_ref[1]) * dinv + b_ref[...]
    h = jnp.maximum(h, 0.0)
    t = lax.dot_general(h, w_ref[...], (((1,), (1,)), ((), ())),
                        preferred_element_type=jnp.float32)
    o_ref[...] = t * dinv


def _tc_last_body(agg_ref, degw_ref, b_ref, o_ref):
    dinv = _dinv_from_degw(degw_ref)
    o_ref[...] = (agg_ref[0] + agg_ref[1]) * dinv + b_ref[...]


def _row_blocks(n):
    for blk in (400, 200, 100, 50, 25, 10, 8):
        if n % blk == 0:
            return blk
    return n


def _make_sc_deg(n_pad, d, rw):
    mesh = plsc.VectorSubcoreMesh(core_axis_name="c", subcore_axis_name="s")
    zrows = n_pad // NS
    orows = n_pad // NS

    def body(dst_hbm, zeros_hbm, ones_hbm, out_hbm, dst_all, ones_v, accd):
        cid = lax.axis_index("c")
        sid = lax.axis_index("s")
        wid = cid * NS + sid
        pltpu.sync_copy(zeros_hbm.at[pl.ds(sid * zrows, zrows)],
                        accd.at[pl.ds(sid * zrows, zrows)])
        pltpu.sync_copy(ones_hbm, ones_v)
        pltpu.sync_copy(dst_hbm.at[pl.ds(wid * rw, rw)], dst_all)
        plsc.subcore_barrier()

        def step(j, c):
            pltpu.sync_copy(ones_v, accd.at[dst_all.at[j]], add=True)
            return c

        lax.fori_loop(0, rw, step, 0)
        plsc.subcore_barrier()
        pltpu.sync_copy(accd.at[pl.ds(sid * orows, orows)],
                        out_hbm.at[cid, pl.ds(sid * orows, orows)])

    return pl.kernel(
        body,
        out_type=jax.ShapeDtypeStruct((NC, n_pad, d), jnp.float32),
        mesh=mesh,
        scratch_types=[
            pltpu.VMEM((rw, CHUNK), jnp.int32),
            pltpu.VMEM((CHUNK, d), jnp.float32),
            pltpu.VMEM_SHARED((n_pad, d), jnp.float32),
        ],
    )


NBUF = 2
IB = 16


def _make_sc_agg(n_pad, d, rw0, rw1):
    mesh = plsc.VectorSubcoreMesh(core_axis_name="c", subcore_axis_name="s")
    zrows = n_pad // NS
    orows = n_pad // NS

    def body(t_hbm, src_hbm, dst_hbm, zeros_hbm, out_hbm,
             src_ib, dst_ib, bufs, acc, sems):
        cid = lax.axis_index("c")
        sid = lax.axis_index("s")
        pltpu.sync_copy(zeros_hbm.at[pl.ds(sid * zrows, zrows)],
                        acc.at[pl.ds(sid * zrows, zrows)])
        plsc.subcore_barrier()

        is0 = cid == 0
        base_chunk = jnp.where(is0, sid * rw0, NS * rw0 + sid * rw1)
        nblk = jnp.where(is0, rw0 // IB, rw1 // IB)

        def block(g, c):
            base = base_chunk + g * IB
            pltpu.sync_copy(src_hbm.at[pl.ds(base, IB)], src_ib)
            pltpu.sync_copy(dst_hbm.at[pl.ds(base, IB)], dst_ib)
            for b in range(NBUF):
                pltpu.async_copy(t_hbm.at[src_ib.at[b]], bufs.at[b],
                                 sems.at[b])
            for j in range(IB):
                b = j % NBUF
                pltpu.make_async_copy(t_hbm.at[src_ib.at[j]], bufs.at[b],
                                      sems.at[b]).wait()
                pltpu.sync_copy(bufs.at[b], acc.at[dst_ib.at[j]], add=True)
                if j + NBUF < IB:
                    pltpu.async_copy(t_hbm.at[src_ib.at[j + NBUF]],
                                     bufs.at[b], sems.at[b])
            return c

        lax.fori_loop(0, nblk, block, 0)
        plsc.subcore_barrier()
        pltpu.sync_copy(acc.at[pl.ds(sid * orows, orows)],
                        out_hbm.at[cid, pl.ds(sid * orows, orows)])

    return pl.kernel(
        body,
        out_type=jax.ShapeDtypeStruct((NC, n_pad, d), jnp.float32),
        mesh=mesh,
        scratch_types=[
            pltpu.VMEM((IB, CHUNK), jnp.int32),
            pltpu.VMEM((IB, CHUNK), jnp.int32),
            pltpu.VMEM((NBUF, CHUNK, d), jnp.float32),
            pltpu.VMEM_SHARED((n_pad, d), jnp.float32),
            pltpu.SemaphoreType.DMA((NBUF,)),
        ],
    )


def kernel(x, edge_index, W1, b1, W2, b2):
    n, d = x.shape
    e = edge_index.shape[1]
    d_hid = W1.shape[0]
    d_out = W2.shape[0]

    nw = NC * NS
    unit = _cdiv(_cdiv(e, CHUNK), NS * 10 * IB) * IB
    rw0, rw1 = 8 * unit, 2 * unit
    rw = (rw0 + rw1) // NC
    e_pad = NS * (rw0 + rw1) * CHUNK
    n_pad = _cdiv(n + 1, 128) * 128

    src = edge_index[0].astype(jnp.int32)
    dst = edge_index[1].astype(jnp.int32)
    src2d = jnp.pad(src, (0, e_pad - e)).reshape(nw * rw, CHUNK)
    dst2d = jnp.pad(dst, (0, e_pad - e),
                    constant_values=n).reshape(nw * rw, CHUNK)

    zeros_d = jnp.zeros((n_pad, d_hid), jnp.float32)
    ones_d = jnp.ones((CHUNK, d_hid), jnp.float32)

    degw = _make_sc_deg(n_pad, d_hid, rw)(dst2d, zeros_d, ones_d)

    sc_agg = _make_sc_agg(n_pad, d_hid, rw0, rw1)

    blk = _row_blocks(n)
    grid = (n // blk,)
    degw_spec = pl.BlockSpec((NC, blk, d_hid), lambda i: (0, i, 0))
    row_spec = pl.BlockSpec((blk, d), lambda i: (i, 0))
    w_spec = pl.BlockSpec((d_hid, d), lambda i: (0, 0))
    b_spec = pl.BlockSpec((1, d_hid), lambda i: (0, 0))
    agg_spec = pl.BlockSpec((NC, blk, d_hid), lambda i: (0, i, 0))

    t1 = pl.pallas_call(
        _tc_first_body,
        grid=grid,
        in_specs=[row_spec, w_spec, degw_spec],
        out_specs=pl.BlockSpec((blk, d_hid), lambda i: (i, 0)),
        out_shape=jax.ShapeDtypeStruct((n, d_hid), jnp.float32),
    )(x, W1, degw)

    agg1 = sc_agg(t1, src2d, dst2d, zeros_d)

    t2 = pl.pallas_call(
        _tc_mid_body,
        grid=grid,
        in_specs=[agg_spec, degw_spec, b_spec,
                  pl.BlockSpec((d_out, d_hid), lambda i: (0, 0))],
        out_specs=pl.BlockSpec((blk, d_out), lambda i: (i, 0)),
        out_shape=jax.ShapeDtypeStruct((n, d_out), jnp.float32),
    )(agg1, degw, b1.reshape(1, d_hid), W2)

    agg2 = sc_agg(t2, src2d, dst2d, zeros_d)

    out = pl.pallas_call(
        _tc_last_body,
        grid=grid,
        in_specs=[agg_spec, degw_spec, b_spec],
        out_specs=pl.BlockSpec((blk, d_out), lambda i: (i, 0)),
        out_shape=jax.ShapeDtypeStruct((n, d_out), jnp.float32),
    )(agg2, degw, b2.reshape(1, d_out))

    return out
